# Initial kernel scaffold; baseline (speedup 1.0000x reference)
import jax
import jax.numpy as jnp
from jax.experimental import pallas as pl


def kernel(morganSMILES, table, W, b):
    raise NotImplementedError("write your pallas kernel here")


# speedup vs baseline: 7.5576x; 7.5576x over previous
import functools

import jax
import jax.numpy as jnp
from jax import lax
from jax.experimental import pallas as pl
from jax.experimental.pallas import tpu as pltpu
from jax.experimental.pallas import tpu_sc as plsc


def _proj_body(t_ref, w_ref, b_ref, o_ref):
    o_ref[...] = (
        jnp.dot(
            t_ref[...],
            w_ref[...],
            preferred_element_type=jnp.float32,
            precision=lax.Precision.HIGHEST,
        )
        + b_ref[...]
    )


def _project_table(table, W, b):
    V, E = table.shape
    D = W.shape[1]
    blk = 2000
    grid = (V + blk - 1) // blk
    return pl.pallas_call(
        _proj_body,
        grid=(grid,),
        in_specs=[
            pl.BlockSpec((blk, E), lambda i: (i, 0)),
            pl.BlockSpec((E, D), lambda i: (0, 0)),
            pl.BlockSpec((1, D), lambda i: (0, 0)),
        ],
        out_specs=pl.BlockSpec((blk, D), lambda i: (i, 0)),
        out_shape=jax.ShapeDtypeStruct((V, D), jnp.float32),
    )(table, W, b.reshape(1, D))


def _gather_rows(P, idx_flat):
    B = idx_flat.shape[0]
    D = P.shape[1]
    window = 128
    mesh = plsc.VectorSubcoreMesh(core_axis_name="c", subcore_axis_name="s")

    @functools.partial(
        pl.kernel,
        out_type=jax.ShapeDtypeStruct((B, D), jnp.float32),
        mesh=mesh,
    )
    def k(p_hbm, i_hbm, o_hbm):
        def body(i_vmem, o_vmem):
            pltpu.sync_copy(p_hbm.at[i_vmem.at[0]], o_vmem)

        pltpu.emit_pipeline(
            body,
            grid=(B // window,),
            in_specs=[pl.BlockSpec((1, window), index_map=lambda i: (0, i))],
            out_specs=[pl.BlockSpec((window, D), index_map=lambda i: (i, 0))],
            core_axis_name=("c", "s"),
            dimension_semantics=(pltpu.PARALLEL,),
        )(i_hbm, o_hbm)

    return k(P, idx_flat.reshape(1, B))


def kernel(morganSMILES, table, W, b):
    Bt, L = morganSMILES.shape
    D = W.shape[1]
    P = _project_table(table, W, b)
    flat_idx = morganSMILES.reshape(-1).astype(jnp.int32)
    out = _gather_rows(P, flat_idx)
    return out.reshape(Bt, L, D)


# speedup vs baseline: 8.4200x; 1.1141x over previous
import functools

import jax
import jax.numpy as jnp
from jax import lax
from jax.experimental import pallas as pl
from jax.experimental.pallas import tpu as pltpu
from jax.experimental.pallas import tpu_sc as plsc


def _proj_body(t_ref, w_ref, b_ref, o_ref):
    o_ref[...] = (
        jnp.dot(
            t_ref[...],
            w_ref[...],
            preferred_element_type=jnp.float32,
            precision=lax.Precision.DEFAULT,
        )
        + b_ref[...]
    )


def _project_table(table, W, b):
    V, E = table.shape
    D = W.shape[1]
    blk = 2000
    grid = (V + blk - 1) // blk
    return pl.pallas_call(
        _proj_body,
        grid=(grid,),
        in_specs=[
            pl.BlockSpec((blk, E), lambda i: (i, 0)),
            pl.BlockSpec((E, D), lambda i: (0, 0)),
            pl.BlockSpec((1, D), lambda i: (0, 0)),
        ],
        out_specs=pl.BlockSpec((blk, D), lambda i: (i, 0)),
        out_shape=jax.ShapeDtypeStruct((V, D), jnp.float32),
    )(table, W, b.reshape(1, D))


def _gather_rows(P, idx_flat):
    B = idx_flat.shape[0]
    D = P.shape[1]
    window = 128
    mesh = plsc.VectorSubcoreMesh(core_axis_name="c", subcore_axis_name="s")

    @functools.partial(
        pl.kernel,
        out_type=jax.ShapeDtypeStruct((B, D), jnp.float32),
        mesh=mesh,
    )
    def k(p_hbm, i_hbm, o_hbm):
        def body(i_vmem, o_vmem):
            pltpu.sync_copy(p_hbm.at[i_vmem.at[0]], o_vmem)

        pltpu.emit_pipeline(
            body,
            grid=(B // window,),
            in_specs=[pl.BlockSpec((1, window), index_map=lambda i: (0, i))],
            out_specs=[pl.BlockSpec((window, D), index_map=lambda i: (i, 0))],
            core_axis_name=("c", "s"),
            dimension_semantics=(pltpu.PARALLEL,),
        )(i_hbm, o_hbm)

    return k(P, idx_flat.reshape(1, B))


def kernel(morganSMILES, table, W, b):
    Bt, L = morganSMILES.shape
    D = W.shape[1]
    P = _project_table(table, W, b)
    flat_idx = morganSMILES.reshape(-1).astype(jnp.int32)
    out = _gather_rows(P, flat_idx)
    return out.reshape(Bt, L, D)


# speedup vs baseline: 9.9321x; 1.1796x over previous
import functools

import jax
import jax.numpy as jnp
from jax import lax
from jax.experimental import pallas as pl
from jax.experimental.pallas import tpu as pltpu
from jax.experimental.pallas import tpu_sc as plsc


def _proj_body(t_ref, w_ref, b_ref, o_ref):
    o_ref[...] = (
        jnp.dot(
            t_ref[...],
            w_ref[...],
            preferred_element_type=jnp.float32,
            precision=lax.Precision.DEFAULT,
        )
        + b_ref[...]
    )


def _project_table(table, W, b):
    V, E = table.shape
    D = W.shape[1]
    blk = 2000
    grid = (V + blk - 1) // blk
    return pl.pallas_call(
        _proj_body,
        grid=(grid,),
        in_specs=[
            pl.BlockSpec((blk, E), lambda i: (i, 0)),
            pl.BlockSpec((E, D), lambda i: (0, 0)),
            pl.BlockSpec((1, D), lambda i: (0, 0)),
        ],
        out_specs=pl.BlockSpec((blk, D), lambda i: (i, 0)),
        out_shape=jax.ShapeDtypeStruct((V, D), jnp.float32),
    )(table, W, b.reshape(1, D))


def _gather_rows(P, idx):
    Bt, L = idx.shape
    D = P.shape[1]
    mesh = plsc.VectorSubcoreMesh(core_axis_name="c", subcore_axis_name="s")

    @functools.partial(
        pl.kernel,
        out_type=jax.ShapeDtypeStruct((Bt, L, D), jnp.float32),
        mesh=mesh,
    )
    def k(p_hbm, i_hbm, o_hbm):
        def body(i_vmem, o_vmem):
            pltpu.sync_copy(p_hbm.at[i_vmem.at[0, 0]], o_vmem.at[0])

        pltpu.emit_pipeline(
            body,
            grid=(Bt,),
            in_specs=[pl.BlockSpec((1, 1, L), index_map=lambda i: (i, 0, 0))],
            out_specs=[pl.BlockSpec((1, L, D), index_map=lambda i: (i, 0, 0))],
            core_axis_name=("c", "s"),
            dimension_semantics=(pltpu.PARALLEL,),
        )(i_hbm, o_hbm)

    return k(P, idx.reshape(Bt, 1, L))


def kernel(morganSMILES, table, W, b):
    Bt, L = morganSMILES.shape
    D = W.shape[1]
    P = _project_table(table, W, b)
    idx = morganSMILES.astype(jnp.int32)
    return _gather_rows(P, idx)


# speedup vs baseline: 22.5992x; 2.2754x over previous
import functools

import jax
import jax.numpy as jnp
from jax import lax
from jax.experimental import pallas as pl
from jax.experimental.pallas import tpu as pltpu
from jax.experimental.pallas import tpu_sc as plsc


def _proj_body(t_ref, w_ref, b_ref, o_ref):
    o_ref[...] = (
        lax.dot_general(
            t_ref[...],
            w_ref[...],
            dimension_numbers=(((0,), (0,)), ((), ())),
            preferred_element_type=jnp.float32,
        )
        + b_ref[...]
    )


def _project_table(tableT, W, b):
    E, V = tableT.shape
    D = W.shape[1]
    blk = 2048
    grid = (V + blk - 1) // blk
    return pl.pallas_call(
        _proj_body,
        grid=(grid,),
        in_specs=[
            pl.BlockSpec((E, blk), lambda i: (0, i)),
            pl.BlockSpec((E, D), lambda i: (0, 0)),
            pl.BlockSpec((1, D), lambda i: (0, 0)),
        ],
        out_specs=pl.BlockSpec((blk, D), lambda i: (i, 0)),
        out_shape=jax.ShapeDtypeStruct((V, D), jnp.float32),
    )(tableT, W, b.reshape(1, D))


def _gather_rows(P, idxT):
    L, Bt = idxT.shape
    D = P.shape[1]
    window = 128
    mesh = plsc.VectorSubcoreMesh(core_axis_name="c", subcore_axis_name="s")

    @functools.partial(
        pl.kernel,
        out_type=jax.ShapeDtypeStruct((L, Bt, D), jnp.float32),
        mesh=mesh,
    )
    def k(p_hbm, i_hbm, o_hbm):
        def body(i_vmem, o_vmem):
            pltpu.sync_copy(p_hbm.at[i_vmem.at[0, 0]], o_vmem.at[0])

        pltpu.emit_pipeline(
            body,
            grid=(L, Bt // window),
            in_specs=[
                pl.BlockSpec((1, 1, window), index_map=lambda l, w: (l, 0, w))
            ],
            out_specs=[
                pl.BlockSpec((1, window, D), index_map=lambda l, w: (l, w, 0))
            ],
            core_axis_name=("c", "s"),
            dimension_semantics=(pltpu.PARALLEL, pltpu.PARALLEL),
        )(i_hbm, o_hbm)

    return k(P, idxT.reshape(L, 1, Bt))


def kernel(morganSMILES, table, W, b):
    Bt, L = morganSMILES.shape
    idxT = morganSMILES.T.astype(jnp.int32)
    P = _project_table(table.T, W, b)
    out = _gather_rows(P, idxT)
    return out.transpose(1, 0, 2)


# speedup vs baseline: 25.2511x; 1.1173x over previous
import functools

import jax
import jax.numpy as jnp
from jax import lax
from jax.experimental import pallas as pl
from jax.experimental.pallas import tpu as pltpu
from jax.experimental.pallas import tpu_sc as plsc


def _proj_body(t_ref, w_ref, b_ref, o_ref):
    o_ref[...] = (
        lax.dot_general(
            t_ref[...],
            w_ref[...],
            dimension_numbers=(((0,), (0,)), ((), ())),
            preferred_element_type=jnp.float32,
        )
        + b_ref[...]
    )


def _project_table(tableT, W, b):
    E, V = tableT.shape
    D = W.shape[1]
    blk = 4096
    grid = (V + blk - 1) // blk
    return pl.pallas_call(
        _proj_body,
        grid=(grid,),
        in_specs=[
            pl.BlockSpec((E, blk), lambda i: (0, i)),
            pl.BlockSpec((E, D), lambda i: (0, 0)),
            pl.BlockSpec((1, D), lambda i: (0, 0)),
        ],
        out_specs=pl.BlockSpec((blk, D), lambda i: (i, 0)),
        out_shape=jax.ShapeDtypeStruct((V, D), jnp.float32),
    )(tableT, W, b.reshape(1, D))


def _gather_rows(P, idxT):
    L, Bt = idxT.shape
    D = P.shape[1]
    window = 256
    mesh = plsc.VectorSubcoreMesh(core_axis_name="c", subcore_axis_name="s")

    @functools.partial(
        pl.kernel,
        out_type=jax.ShapeDtypeStruct((L, Bt, D), jnp.float32),
        mesh=mesh,
    )
    def k(p_hbm, i_hbm, o_hbm):
        def body(i_vmem, o_vmem):
            pltpu.sync_copy(p_hbm.at[i_vmem.at[0, 0]], o_vmem.at[0])

        pltpu.emit_pipeline(
            body,
            grid=(L, Bt // window),
            in_specs=[
                pl.BlockSpec((1, 1, window), index_map=lambda l, w: (l, 0, w))
            ],
            out_specs=[
                pl.BlockSpec((1, window, D), index_map=lambda l, w: (l, w, 0))
            ],
            core_axis_name=("c", "s"),
            dimension_semantics=(pltpu.PARALLEL, pltpu.PARALLEL),
        )(i_hbm, o_hbm)

    return k(P, idxT.reshape(L, 1, Bt))


def kernel(morganSMILES, table, W, b):
    Bt, L = morganSMILES.shape
    idxT = morganSMILES.T.astype(jnp.int32)
    P = _project_table(table.T, W, b)
    out = _gather_rows(P, idxT)
    return out.transpose(1, 0, 2)


# speedup vs baseline: 25.7625x; 1.0203x over previous
import functools

import jax
import jax.numpy as jnp
from jax import lax
from jax.experimental import pallas as pl
from jax.experimental.pallas import tpu as pltpu
from jax.experimental.pallas import tpu_sc as plsc


def _proj_body(t_ref, w_ref, b_ref, o_ref):
    o_ref[...] = (
        lax.dot_general(
            t_ref[...],
            w_ref[...],
            dimension_numbers=(((0,), (0,)), ((), ())),
            preferred_element_type=jnp.float32,
        )
        + b_ref[...]
    )


def _project_table(tableT, W, b):
    E, V = tableT.shape
    D = W.shape[1]
    blk = 8192
    grid = (V + blk - 1) // blk
    return pl.pallas_call(
        _proj_body,
        grid=(grid,),
        in_specs=[
            pl.BlockSpec((E, blk), lambda i: (0, i)),
            pl.BlockSpec((E, D), lambda i: (0, 0)),
            pl.BlockSpec((1, D), lambda i: (0, 0)),
        ],
        out_specs=pl.BlockSpec((blk, D), lambda i: (i, 0)),
        out_shape=jax.ShapeDtypeStruct((V, D), jnp.float32),
    )(tableT, W, b.reshape(1, D))


def _gather_rows(P, idxT):
    L, Bt = idxT.shape
    D = P.shape[1]
    window = 256
    mesh = plsc.VectorSubcoreMesh(core_axis_name="c", subcore_axis_name="s")

    @functools.partial(
        pl.kernel,
        out_type=jax.ShapeDtypeStruct((L, Bt, D), jnp.float32),
        mesh=mesh,
    )
    def k(p_hbm, i_hbm, o_hbm):
        def body(i_vmem, o_vmem):
            pltpu.sync_copy(p_hbm.at[i_vmem.at[0, 0]], o_vmem.at[0])

        pltpu.emit_pipeline(
            body,
            grid=(L, Bt // window),
            in_specs=[
                pl.BlockSpec((1, 1, window), index_map=lambda l, w: (l, 0, w))
            ],
            out_specs=[
                pl.BlockSpec((1, window, D), index_map=lambda l, w: (l, w, 0))
            ],
            core_axis_name=("c", "s"),
            dimension_semantics=(pltpu.PARALLEL, pltpu.PARALLEL),
        )(i_hbm, o_hbm)

    return k(P, idxT.reshape(L, 1, Bt))


def kernel(morganSMILES, table, W, b):
    Bt, L = morganSMILES.shape
    idxT = morganSMILES.T.astype(jnp.int32)
    P = _project_table(table.T, W, b)
    out = _gather_rows(P, idxT)
    return out.transpose(1, 0, 2)


# speedup vs baseline: 27.8976x; 1.0829x over previous
import functools

import jax
import jax.numpy as jnp
from jax import lax
from jax.experimental import pallas as pl
from jax.experimental.pallas import tpu as pltpu
from jax.experimental.pallas import tpu_sc as plsc


def _proj_body(t_ref, w_ref, b_ref, o_ref):
    o_ref[...] = (
        lax.dot_general(
            t_ref[...],
            w_ref[...],
            dimension_numbers=(((0,), (0,)), ((), ())),
            preferred_element_type=jnp.float32,
        )
        + b_ref[...]
    )


def _project_table(tableT, W, b):
    E, V = tableT.shape
    D = W.shape[1]
    blk = 8192
    grid = (V + blk - 1) // blk
    return pl.pallas_call(
        _proj_body,
        grid=(grid,),
        in_specs=[
            pl.BlockSpec((E, blk), lambda i: (0, i)),
            pl.BlockSpec((E, D), lambda i: (0, 0)),
            pl.BlockSpec((1, D), lambda i: (0, 0)),
        ],
        out_specs=pl.BlockSpec((blk, D), lambda i: (i, 0)),
        out_shape=jax.ShapeDtypeStruct((V, D), jnp.float32),
    )(tableT, W, b.reshape(1, D))


def _gather_rows(P, idx_flat):
    (B,) = idx_flat.shape
    D = P.shape[1]
    mesh = plsc.VectorSubcoreMesh(core_axis_name="c", subcore_axis_name="s")
    nw = mesh.num_cores * mesh.num_subcores
    b_per_w = B // nw

    chunk = 400
    nchunk = b_per_w // chunk

    @functools.partial(
        pl.kernel,
        out_type=jax.ShapeDtypeStruct((B, D), jnp.float32),
        mesh=mesh,
        scratch_types=[
            pltpu.VMEM((b_per_w,), jnp.int32),
            pltpu.VMEM((chunk, D), jnp.float32),
            pltpu.VMEM((chunk, D), jnp.float32),
            pltpu.SemaphoreType.DMA,
            pltpu.SemaphoreType.DMA,
            pltpu.SemaphoreType.DMA,
            pltpu.SemaphoreType.DMA,
        ],
    )
    def k(p_hbm, i_hbm, o_hbm, idx_v, rows_a, rows_b, gs_a, gs_b, ws_a, ws_b):
        wid = lax.axis_index("s") * mesh.num_cores + lax.axis_index("c")
        base = wid * b_per_w
        pltpu.sync_copy(i_hbm.at[pl.ds(base, b_per_w)], idx_v)

        bufs = (rows_a, rows_b)
        gsems = (gs_a, gs_b)
        wsems = (ws_a, ws_b)

        def gather(j, p):
            return pltpu.async_copy(
                p_hbm.at[idx_v.at[pl.ds(j * chunk, chunk)]], bufs[p], gsems[p]
            )

        def writeback(j, p):
            return pltpu.async_copy(
                bufs[p], o_hbm.at[pl.ds(base + j * chunk, chunk)], wsems[p]
            )

        g_h = [None, None]
        w_h = [None, None]
        g_h[0] = gather(0, 0)
        for j in range(nchunk):
            p = j % 2
            g_h[p].wait()
            if j + 1 < nchunk:
                if w_h[1 - p] is not None:
                    w_h[1 - p].wait()
                g_h[1 - p] = gather(j + 1, 1 - p)
            w_h[p] = writeback(j, p)
        for h in w_h:
            if h is not None:
                h.wait()

    return k(P, idx_flat)


def kernel(morganSMILES, table, W, b):
    Bt, L = morganSMILES.shape
    D = W.shape[1]
    idx_flat = morganSMILES.T.astype(jnp.int32).reshape(-1)
    P = _project_table(table.T, W, b)
    out = _gather_rows(P, idx_flat)
    return out.reshape(L, Bt, D).transpose(1, 0, 2)


# speedup vs baseline: 28.2610x; 1.0130x over previous
import functools

import jax
import jax.numpy as jnp
from jax import lax
from jax.experimental import pallas as pl
from jax.experimental.pallas import tpu as pltpu
from jax.experimental.pallas import tpu_sc as plsc


def _proj_body(t_ref, w_ref, b_ref, o_ref):
    o_ref[...] = (
        lax.dot_general(
            t_ref[...],
            w_ref[...],
            dimension_numbers=(((0,), (0,)), ((), ())),
            preferred_element_type=jnp.float32,
        )
        + b_ref[...]
    )


def _project_table(tableT, W, b):
    E, V = tableT.shape
    D = W.shape[1]
    blk = 12800
    grid = (V + blk - 1) // blk
    return pl.pallas_call(
        _proj_body,
        grid=(grid,),
        in_specs=[
            pl.BlockSpec((E, blk), lambda i: (0, i)),
            pl.BlockSpec((E, D), lambda i: (0, 0)),
            pl.BlockSpec((1, D), lambda i: (0, 0)),
        ],
        out_specs=pl.BlockSpec((blk, D), lambda i: (i, 0)),
        out_shape=jax.ShapeDtypeStruct((V, D), jnp.float32),
    )(tableT, W, b.reshape(1, D))


def _gather_rows(P, idx_flat):
    (B,) = idx_flat.shape
    D = P.shape[1]
    mesh = plsc.VectorSubcoreMesh(core_axis_name="c", subcore_axis_name="s")
    nw = mesh.num_cores * mesh.num_subcores
    b_per_w = B // nw

    chunk = 400
    nchunk = b_per_w // chunk

    @functools.partial(
        pl.kernel,
        out_type=jax.ShapeDtypeStruct((B, D), jnp.float32),
        mesh=mesh,
        scratch_types=[
            pltpu.VMEM((b_per_w,), jnp.int32),
            pltpu.VMEM((chunk, D), jnp.float32),
            pltpu.VMEM((chunk, D), jnp.float32),
            pltpu.SemaphoreType.DMA,
            pltpu.SemaphoreType.DMA,
            pltpu.SemaphoreType.DMA,
            pltpu.SemaphoreType.DMA,
        ],
    )
    def k(p_hbm, i_hbm, o_hbm, idx_v, rows_a, rows_b, gs_a, gs_b, ws_a, ws_b):
        wid = lax.axis_index("s") * mesh.num_cores + lax.axis_index("c")
        base = wid * b_per_w
        pltpu.sync_copy(i_hbm.at[pl.ds(base, b_per_w)], idx_v)

        bufs = (rows_a, rows_b)
        gsems = (gs_a, gs_b)
        wsems = (ws_a, ws_b)

        def gather(j, p):
            return pltpu.async_copy(
                p_hbm.at[idx_v.at[pl.ds(j * chunk, chunk)]], bufs[p], gsems[p]
            )

        def writeback(j, p):
            return pltpu.async_copy(
                bufs[p], o_hbm.at[pl.ds(base + j * chunk, chunk)], wsems[p]
            )

        g_h = [None, None]
        w_h = [None, None]
        g_h[0] = gather(0, 0)
        for j in range(nchunk):
            p = j % 2
            g_h[p].wait()
            if j + 1 < nchunk:
                if w_h[1 - p] is not None:
                    w_h[1 - p].wait()
                g_h[1 - p] = gather(j + 1, 1 - p)
            w_h[p] = writeback(j, p)
        for h in w_h:
            if h is not None:
                h.wait()

    return k(P, idx_flat)


def kernel(morganSMILES, table, W, b):
    Bt, L = morganSMILES.shape
    D = W.shape[1]
    idx_flat = morganSMILES.T.astype(jnp.int32).reshape(-1)
    P = _project_table(table.T, W, b)
    out = _gather_rows(P, idx_flat)
    return out.reshape(L, Bt, D).transpose(1, 0, 2)


# speedup vs baseline: 28.8361x; 1.0203x over previous
import functools

import jax
import jax.numpy as jnp
from jax import lax
from jax.experimental import pallas as pl
from jax.experimental.pallas import tpu as pltpu
from jax.experimental.pallas import tpu_sc as plsc


def _proj_body(t_ref, w_ref, b_ref, o_ref):
    o_ref[...] = (
        lax.dot_general(
            t_ref[...],
            w_ref[...],
            dimension_numbers=(((0,), (0,)), ((), ())),
            preferred_element_type=jnp.float32,
        )
        + b_ref[...]
    )


def _project_table(tableT, W, b):
    E, V = tableT.shape
    D = W.shape[1]
    blk = 12800
    grid = (V + blk - 1) // blk
    return pl.pallas_call(
        _proj_body,
        grid=(grid,),
        in_specs=[
            pl.BlockSpec((E, blk), lambda i: (0, i)),
            pl.BlockSpec((E, D), lambda i: (0, 0)),
            pl.BlockSpec((1, D), lambda i: (0, 0)),
        ],
        out_specs=pl.BlockSpec((blk, D), lambda i: (i, 0)),
        out_shape=jax.ShapeDtypeStruct((V, D), jnp.float32),
    )(tableT, W, b.reshape(1, D))


def _gather_rows(P, idx_flat):
    (B,) = idx_flat.shape
    D = P.shape[1]
    mesh = plsc.VectorSubcoreMesh(core_axis_name="c", subcore_axis_name="s")
    nw = mesh.num_cores * mesh.num_subcores
    b_per_w = B // nw

    chunk = 256
    nbuf = 3
    nchunk = b_per_w // chunk

    @functools.partial(
        pl.kernel,
        out_type=jax.ShapeDtypeStruct((B, D), jnp.float32),
        mesh=mesh,
        scratch_types=(
            [pltpu.VMEM((b_per_w,), jnp.int32)]
            + [pltpu.VMEM((chunk, D), jnp.float32)] * nbuf
            + [pltpu.SemaphoreType.DMA] * (2 * nbuf)
        ),
    )
    def k(p_hbm, i_hbm, o_hbm, idx_v, *rest):
        bufs = rest[:nbuf]
        gsems = rest[nbuf : 2 * nbuf]
        wsems = rest[2 * nbuf :]
        wid = lax.axis_index("s") * mesh.num_cores + lax.axis_index("c")
        base = wid * b_per_w
        pltpu.sync_copy(i_hbm.at[pl.ds(base, b_per_w)], idx_v)

        def gather(j, p):
            return pltpu.async_copy(
                p_hbm.at[idx_v.at[pl.ds(j * chunk, chunk)]], bufs[p], gsems[p]
            )

        def writeback(j, p):
            return pltpu.async_copy(
                bufs[p], o_hbm.at[pl.ds(base + j * chunk, chunk)], wsems[p]
            )

        g_h = [None] * nbuf
        w_h = [None] * nbuf
        for j in range(min(nbuf, nchunk)):
            g_h[j] = gather(j, j)
        for j in range(nchunk):
            p = j % nbuf
            g_h[p].wait()
            w_h[p] = writeback(j, p)
            nxt = j + nbuf
            if nxt < nchunk:
                w_h[p].wait()
                g_h[p] = gather(nxt, p)
        for h in w_h:
            if h is not None:
                h.wait()

    return k(P, idx_flat)


def kernel(morganSMILES, table, W, b):
    Bt, L = morganSMILES.shape
    D = W.shape[1]
    idx_flat = morganSMILES.T.astype(jnp.int32).reshape(-1)
    P = _project_table(table.T, W, b)
    out = _gather_rows(P, idx_flat)
    return out.reshape(L, Bt, D).transpose(1, 0, 2)
